# Initial kernel scaffold; baseline (speedup 1.0000x reference)
import functools

import jax
import jax.numpy as jnp
from jax import lax
from jax.experimental import pallas as pl
from jax.experimental.pallas import tpu as pltpu
from jax.experimental.pallas import tpu_sc as plsc

N_NODES = 10000
N_EDGES = 320000
D_FEAT = 128

NC = 2
NS = 16
NW = NC * NS

E_PER_W = N_EDGES // NW
CHUNK = 80
N_CHUNKS = E_PER_W // CHUNK
ROWS_PER_TILE = N_NODES // NS
ZROWS = 25


def _sc_aggregate_body(x_hbm, ra_hbm, rb_hbm, out_hbm,
                       idx_a, idx_b, rows, zbuf, acc, sem):
    cid = lax.axis_index("c")
    sid = lax.axis_index("s")
    wid = cid * NS + sid

    def fill_zero(i, _):
        r = i // 8
        j = i % 8
        zbuf[r, pl.ds(j * 16, 16)] = jnp.zeros((16,), jnp.float32)
        return 0

    lax.fori_loop(0, ZROWS * 8, fill_zero, 0)

    row0 = sid * ROWS_PER_TILE

    def zero_acc(k, _):
        pltpu.sync_copy(zbuf, acc.at[pl.ds(row0 + k * ZROWS, ZROWS)])
        return 0

    lax.fori_loop(0, ROWS_PER_TILE // ZROWS, zero_acc, 0)

    plsc.subcore_barrier()

    ebase = wid * E_PER_W

    def edge_step(i, _):
        base = ebase + i * CHUNK
        pltpu.sync_copy(ra_hbm.at[pl.ds(base, CHUNK)], idx_a)
        pltpu.sync_copy(rb_hbm.at[pl.ds(base, CHUNK)], idx_b)
        pltpu.async_copy(x_hbm.at[idx_a], rows, sem).wait()
        pltpu.sync_copy(rows, acc.at[idx_b], add=True)
        return 0

    lax.fori_loop(0, N_CHUNKS, edge_step, 0)

    plsc.subcore_barrier()

    obase = cid * N_NODES + row0
    pltpu.sync_copy(acc.at[pl.ds(row0, ROWS_PER_TILE)],
                    out_hbm.at[pl.ds(obase, ROWS_PER_TILE)])


def _sc_aggregate(X, ref_a, ref_b):
    mesh = plsc.VectorSubcoreMesh(core_axis_name="c", subcore_axis_name="s",
                                  num_cores=NC, num_subcores=NS)
    f = pl.kernel(
        _sc_aggregate_body,
        out_type=jax.ShapeDtypeStruct((NC * N_NODES, D_FEAT), jnp.float32),
        mesh=mesh,
        scratch_types=[
            pltpu.VMEM((CHUNK,), jnp.int32),
            pltpu.VMEM((CHUNK,), jnp.int32),
            pltpu.VMEM((CHUNK, D_FEAT), jnp.float32),
            pltpu.VMEM((ZROWS, D_FEAT), jnp.float32),
            pltpu.VMEM_SHARED((N_NODES, D_FEAT), jnp.float32),
            pltpu.SemaphoreType.DMA,
        ],
    )
    return f(X, ref_a, ref_b)


def _mlp_body(x_ref, a0_ref, a1_ref, wh_ref, bh_ref, wo_ref, bo_ref, o_ref):
    xa = x_ref[...] + a0_ref[...] + a1_ref[...]
    h = jnp.dot(xa, wh_ref[...], preferred_element_type=jnp.float32)
    h = jnp.maximum(h + bh_ref[...], 0.0)
    o = jnp.dot(h, wo_ref[...], preferred_element_type=jnp.float32)
    o_ref[...] = jnp.maximum(o + bo_ref[...], 0.0)


def _mlp(X, agg, W_hidden, b_hidden, W_out, b_out):
    R = 1000
    n_blocks = N_NODES // R
    full = lambda i: (0, 0)
    return pl.pallas_call(
        _mlp_body,
        grid=(n_blocks,),
        in_specs=[
            pl.BlockSpec((R, D_FEAT), lambda i: (i, 0)),
            pl.BlockSpec((R, D_FEAT), lambda i: (i, 0)),
            pl.BlockSpec((R, D_FEAT), lambda i: (i + n_blocks, 0)),
            pl.BlockSpec((D_FEAT, D_FEAT), full),
            pl.BlockSpec((1, D_FEAT), full),
            pl.BlockSpec((D_FEAT, D_FEAT), full),
            pl.BlockSpec((1, D_FEAT), full),
        ],
        out_specs=pl.BlockSpec((R, D_FEAT), lambda i: (i, 0)),
        out_shape=jax.ShapeDtypeStruct((N_NODES, D_FEAT), jnp.float32),
    )(X, agg, agg, W_hidden, b_hidden, W_out, b_out)


@jax.jit
def kernel(X, ref_a, ref_b, W_hidden, b_hidden, W_out, b_out):
    ref_a = ref_a.astype(jnp.int32)
    ref_b = ref_b.astype(jnp.int32)
    agg = _sc_aggregate(X, ref_a, ref_b)
    return _mlp(X, agg, W_hidden, b_hidden.reshape(1, -1),
                W_out, b_out.reshape(1, -1))


# speedup vs baseline: 5.5722x; 5.5722x over previous
import jax
import jax.numpy as jnp
from jax import lax
from jax.experimental import pallas as pl
from jax.experimental.pallas import tpu as pltpu
from jax.experimental.pallas import tpu_sc as plsc

N_NODES = 10000
N_EDGES = 320000
D_FEAT = 128

NC = 2
NS = 16
NW = NC * NS

N_PAD = 10240
E_PER_W = N_EDGES // NW
CHUNK = 80
N_CHUNKS = E_PER_W // CHUNK
ROWS_PER_TILE = N_PAD // NS
ZROWS = 32


def _sc_aggregate_body(x_hbm, ra_hbm, rb_hbm, out_hbm,
                       idx_a, idx_b, rows, zbuf, acc, sem):
    cid = lax.axis_index("c")
    sid = lax.axis_index("s")
    wid = cid * NS + sid

    def fill_zero(i, _):
        r = i // 8
        j = i % 8
        zbuf[r, pl.ds(j * 16, 16)] = jnp.zeros((16,), jnp.float32)
        return 0

    lax.fori_loop(0, ZROWS * 8, fill_zero, 0)

    row0 = sid * ROWS_PER_TILE

    def zero_acc(k, _):
        pltpu.sync_copy(zbuf, acc.at[pl.ds(row0 + k * ZROWS, ZROWS)])
        return 0

    lax.fori_loop(0, ROWS_PER_TILE // ZROWS, zero_acc, 0)

    plsc.subcore_barrier()

    ebase = wid * E_PER_W

    def edge_step(i, _):
        base = ebase + i * CHUNK
        pltpu.sync_copy(ra_hbm.at[pl.ds(base, CHUNK)], idx_a)
        pltpu.sync_copy(rb_hbm.at[pl.ds(base, CHUNK)], idx_b)
        pltpu.async_copy(x_hbm.at[idx_a], rows, sem).wait()
        pltpu.sync_copy(rows, acc.at[idx_b], add=True)
        return 0

    lax.fori_loop(0, N_CHUNKS, edge_step, 0)

    plsc.subcore_barrier()

    obase = cid * N_PAD + row0
    pltpu.sync_copy(acc.at[pl.ds(row0, ROWS_PER_TILE)],
                    out_hbm.at[pl.ds(obase, ROWS_PER_TILE)])


def _sc_aggregate(X, ref_a, ref_b):
    mesh = plsc.VectorSubcoreMesh(core_axis_name="c", subcore_axis_name="s",
                                  num_cores=NC, num_subcores=NS)
    f = pl.kernel(
        _sc_aggregate_body,
        out_type=jax.ShapeDtypeStruct((NC * N_PAD, D_FEAT), jnp.float32),
        mesh=mesh,
        scratch_types=[
            pltpu.VMEM((CHUNK,), jnp.int32),
            pltpu.VMEM((CHUNK,), jnp.int32),
            pltpu.VMEM((CHUNK, D_FEAT), jnp.float32),
            pltpu.VMEM((ZROWS, D_FEAT), jnp.float32),
            pltpu.VMEM_SHARED((N_PAD, D_FEAT), jnp.float32),
            pltpu.SemaphoreType.DMA,
        ],
    )
    return f(X, ref_a, ref_b)


def _mlp_body(x_ref, a0_ref, a1_ref, wh_ref, bh_ref, wo_ref, bo_ref, o_ref):
    xa = x_ref[...] + a0_ref[0] + a1_ref[0]
    h = jnp.dot(xa, wh_ref[...], preferred_element_type=jnp.float32)
    h = jnp.maximum(h + bh_ref[...], 0.0)
    o = jnp.dot(h, wo_ref[...], preferred_element_type=jnp.float32)
    o_ref[...] = jnp.maximum(o + bo_ref[...], 0.0)


def _mlp(X, agg3, W_hidden, b_hidden, W_out, b_out):
    R = 1000
    n_blocks = N_NODES // R
    full = lambda i: (0, 0)
    return pl.pallas_call(
        _mlp_body,
        grid=(n_blocks,),
        in_specs=[
            pl.BlockSpec((R, D_FEAT), lambda i: (i, 0)),
            pl.BlockSpec((1, R, D_FEAT), lambda i: (0, i, 0)),
            pl.BlockSpec((1, R, D_FEAT), lambda i: (1, i, 0)),
            pl.BlockSpec((D_FEAT, D_FEAT), full),
            pl.BlockSpec((1, D_FEAT), full),
            pl.BlockSpec((D_FEAT, D_FEAT), full),
            pl.BlockSpec((1, D_FEAT), full),
        ],
        out_specs=pl.BlockSpec((R, D_FEAT), lambda i: (i, 0)),
        out_shape=jax.ShapeDtypeStruct((N_NODES, D_FEAT), jnp.float32),
    )(X, agg3, agg3, W_hidden, b_hidden, W_out, b_out)


@jax.jit
def kernel(X, ref_a, ref_b, W_hidden, b_hidden, W_out, b_out):
    ref_a = ref_a.astype(jnp.int32)
    ref_b = ref_b.astype(jnp.int32)
    agg = _sc_aggregate(X, ref_a, ref_b)
    agg3 = agg.reshape(NC, N_PAD, D_FEAT)
    return _mlp(X, agg3, W_hidden, b_hidden.reshape(1, -1),
                W_out, b_out.reshape(1, -1))
